# baseline (speedup 1.0000x reference)
import functools

import jax
import jax.numpy as jnp
from jax import lax
from jax.experimental import pallas as pl
from jax.experimental.pallas import tpu as pltpu
from jax.experimental.pallas import tpu_sc as plsc

N_POINTS = 16384
N1 = 4096
N2 = 1024
K_NBR = 32
EPS = 1e-5


def _fps_body(n, px_ref, py_ref, pz_ref, pxs_ref, pys_ref, pzs_ref,
              qx_ref, qy_ref, qz_ref):
    R = px_ref.shape[0]
    px = px_ref[...]
    py = py_ref[...]
    pz = pz_ref[...]
    row_i = lax.broadcasted_iota(jnp.int32, (R, 128), 0)
    col_i = lax.broadcasted_iota(jnp.int32, (R, 128), 1)
    flat_i = row_i * 128 + col_i

    lx0 = pxs_ref[0]
    ly0 = pys_ref[0]
    lz0 = pzs_ref[0]
    qx_ref[0] = lx0
    qy_ref[0] = ly0
    qz_ref[0] = lz0

    dists0 = jnp.full((R, 128), jnp.inf, dtype=jnp.float32)

    def body(i, state):
        dists, lx, ly, lz = state
        dx = px - lx
        dy = py - ly
        dz = pz - lz
        d = (dx * dx + dy * dy) + dz * dz
        dists = jnp.minimum(dists, d)
        m = jnp.max(dists)
        cand = jnp.where(dists == m, flat_i, jnp.int32(1 << 30))
        nxt = jnp.min(cand)
        nlx = pxs_ref[nxt]
        nly = pys_ref[nxt]
        nlz = pzs_ref[nxt]
        qx_ref[i] = nlx
        qy_ref[i] = nly
        qz_ref[i] = nlz
        return (dists, nlx, nly, nlz)

    lax.fori_loop(1, n, body, (dists0, lx0, ly0, lz0), unroll=False)


def _fps_qpos(pos, n):
    N = pos.shape[0]
    R = N // 128
    planes = pos.T.reshape(3, R, 128)
    flat = pos.T
    q = pl.pallas_call(
        functools.partial(_fps_body, n),
        in_specs=[
            pl.BlockSpec(memory_space=pltpu.VMEM),
            pl.BlockSpec(memory_space=pltpu.VMEM),
            pl.BlockSpec(memory_space=pltpu.VMEM),
            pl.BlockSpec(memory_space=pltpu.SMEM),
            pl.BlockSpec(memory_space=pltpu.SMEM),
            pl.BlockSpec(memory_space=pltpu.SMEM),
        ],
        out_specs=[pl.BlockSpec(memory_space=pltpu.SMEM)] * 3,
        out_shape=[jax.ShapeDtypeStruct((n,), jnp.float32)] * 3,
    )(planes[0], planes[1], planes[2], flat[0], flat[1], flat[2])
    return jnp.stack(q, axis=1)


def _topk_body(r2, K, N, q_ref, p_ref, nbr_ref, val_ref, key_ref):
    Q = q_ref.shape[0]
    q8 = q_ref[...]
    p8 = p_ref[...]
    qp = jnp.dot(q8, p8, preferred_element_type=jnp.float32)
    qn = jnp.sum(q8 * q8, axis=1, keepdims=True)
    pn = jnp.sum(p8 * p8, axis=0, keepdims=True)
    d2 = (qn + pn) - 2.0 * qp
    key_ref[...] = jnp.where(d2 <= r2, d2, jnp.inf)
    flat = lax.broadcasted_iota(jnp.int32, (Q, N), 1)
    lane = lax.broadcasted_iota(jnp.int32, (Q, 128), 1)

    def rnd(k, carry):
        nbr_acc, val_acc = carry
        kv = key_ref[...]
        m = jnp.min(kv, axis=1, keepdims=True)
        cand = jnp.where(kv == m, flat, jnp.int32(N))
        nxt = jnp.min(cand, axis=1, keepdims=True)
        key_ref[...] = jnp.where(flat == nxt, jnp.inf, kv)
        nbr_acc = jnp.where(lane == k, nxt, nbr_acc)
        val_acc = jnp.where(lane == k, (m < jnp.inf).astype(jnp.int32), val_acc)
        return nbr_acc, val_acc

    z = jnp.zeros((Q, 128), jnp.int32)
    nbr_acc, val_acc = lax.fori_loop(0, K, rnd, (z, z), unroll=False)
    nbr_ref[...] = nbr_acc
    val_ref[...] = val_acc


def _radius_topk(qpos, pos, r2, Q=128):
    n, N = qpos.shape[0], pos.shape[0]
    q8 = jnp.pad(qpos, ((0, 0), (0, 5)))
    p8 = jnp.pad(pos.T, ((0, 5), (0, 0)))
    grid = n // Q
    nbr, val = pl.pallas_call(
        functools.partial(_topk_body, r2, K_NBR, N),
        grid=(grid,),
        in_specs=[
            pl.BlockSpec((Q, 8), lambda i: (i, 0)),
            pl.BlockSpec((8, N), lambda i: (0, 0)),
        ],
        out_specs=[
            pl.BlockSpec((Q, 128), lambda i: (i, 0)),
            pl.BlockSpec((Q, 128), lambda i: (i, 0)),
        ],
        out_shape=[
            jax.ShapeDtypeStruct((n, 128), jnp.int32),
            jax.ShapeDtypeStruct((n, 128), jnp.int32),
        ],
        scratch_shapes=[pltpu.VMEM((Q, N), jnp.float32)],
    )(q8, p8)
    return nbr[:, :K_NBR], val[:, :K_NBR] != 0


def _sc_gather(table, idx):
    B = idx.shape[0]
    D = table.shape[1]
    info = plsc.get_sparse_core_info()
    NC, NS = info.num_cores, info.num_subcores
    NW = NC * NS
    b_per_w = B // NW
    CH = 128
    n_ch = b_per_w // CH
    idx2d = idx.reshape(B // CH, CH)
    mesh = plsc.VectorSubcoreMesh(core_axis_name="c", subcore_axis_name="s")

    @functools.partial(
        pl.kernel, mesh=mesh,
        compiler_params=pltpu.CompilerParams(use_tc_tiling_on_sc=False),
        out_type=jax.ShapeDtypeStruct((B, D), jnp.float32),
        scratch_types=[
            pltpu.VMEM((n_ch, CH), jnp.int32),
            pltpu.VMEM((b_per_w, D), jnp.float32),
            pltpu.SemaphoreType.DMA,
        ],
    )
    def k(table_hbm, idx_hbm, out_hbm, idx_v, rows_v, sem):
        wid = lax.axis_index("s") * NC + lax.axis_index("c")
        pltpu.sync_copy(idx_hbm.at[pl.ds(wid * n_ch, n_ch)], idx_v)

        def body(j, carry):
            pltpu.async_copy(table_hbm.at[idx_v.at[j]],
                             rows_v.at[pl.ds(j * CH, CH)], sem).wait()
            return carry

        lax.fori_loop(0, n_ch, body, 0)
        pltpu.sync_copy(rows_v, out_hbm.at[pl.ds(wid * b_per_w, b_per_w)])

    return k(table, idx2d)


def _sa(x, pos, n_samples, r, W1, b1, W2, b2, W3, b3):
    qpos = _fps_qpos(pos, n_samples)
    nbr, valid = _radius_topk(qpos, pos, r * r)
    C = x.shape[1]
    Dp = ((C + 3 + 15) // 16) * 16
    table = jnp.pad(jnp.concatenate([x, pos], axis=1),
                    ((0, 0), (0, Dp - C - 3)))
    rows = _sc_gather(table, nbr.reshape(-1))
    xj = rows[:, :C].reshape(n_samples, K_NBR, C)
    posj = rows[:, C:C + 3].reshape(n_samples, K_NBR, 3)
    rel = posj - qpos[:, None, :]
    msg = jnp.concatenate([xj, rel], axis=-1).reshape(n_samples * K_NBR, -1)
    maskf = valid.reshape(n_samples * K_NBR, 1).astype(jnp.float32)
    pad = Dp - msg.shape[1] - 1
    msg = jnp.concatenate(
        [msg, jnp.zeros((msg.shape[0], pad), jnp.float32), maskf], axis=1)
    h = _mlp_sa(msg, W1, b1, W2, b2, W3, b3)
    return h, qpos


def _mlp_sa_body(K, msg_ref, W1_ref, b1_ref, W2_ref, b2_ref,
                 W3_ref, b3_ref, out_ref,
                 s1_ref, q1_ref, s2_ref, q2_ref, sm_ref):
    CH, Dp = msg_ref.shape
    F3 = W3_ref.shape[1]
    p = pl.program_id(0)
    j = pl.program_id(1)

    @pl.when(jnp.logical_and(p == 0, j == 0))
    def _init():
        s1_ref[...] = jnp.zeros_like(s1_ref)
        q1_ref[...] = jnp.zeros_like(q1_ref)
        s2_ref[...] = jnp.zeros_like(s2_ref)
        q2_ref[...] = jnp.zeros_like(q2_ref)
        sm_ref[...] = jnp.zeros_like(sm_ref)

    mc = msg_ref[...]
    mk = mc[:, Dp - 1:Dp]

    @pl.when(p == 0)
    def _ph1():
        pre = jnp.dot(mc, W1_ref[...], preferred_element_type=jnp.float32) \
            + b1_ref[...]
        wp = mk * pre
        s1_ref[...] += jnp.sum(wp, axis=0, keepdims=True)
        q1_ref[...] += jnp.sum(wp * pre, axis=0, keepdims=True)
        sm_ref[...] += jnp.sum(mk, keepdims=True).reshape(1, 1)

    @pl.when(p > 0)
    def _ph23():
        sm = sm_ref[0, 0]
        mean1 = s1_ref[...] / sm
        inv1 = 1.0 / jnp.sqrt(q1_ref[...] / sm - mean1 * mean1 + EPS)
        pre1 = jnp.dot(mc, W1_ref[...], preferred_element_type=jnp.float32) \
            + b1_ref[...]
        h1 = jax.nn.sigmoid((pre1 - mean1) * inv1)
        pre2 = jnp.dot(h1, W2_ref[...], preferred_element_type=jnp.float32) \
            + b2_ref[...]

        @pl.when(p == 1)
        def _acc2():
            wp = mk * pre2
            s2_ref[...] += jnp.sum(wp, axis=0, keepdims=True)
            q2_ref[...] += jnp.sum(wp * pre2, axis=0, keepdims=True)

        @pl.when(p == 2)
        def _ph3():
            mean2 = s2_ref[...] / sm
            inv2 = 1.0 / jnp.sqrt(q2_ref[...] / sm - mean2 * mean2 + EPS)
            h2 = jax.nn.sigmoid((pre2 - mean2) * inv2)
            h3 = jnp.dot(h2, W3_ref[...],
                         preferred_element_type=jnp.float32) + b3_ref[...]
            h3 = jnp.where(mk > 0.0, h3, -jnp.inf)
            pooled = jnp.max(h3.reshape(CH // K, K, F3), axis=1)
            out_ref[...] = jnp.maximum(pooled, 0.0)


def _mlp_sa(msg, W1, b1, W2, b2, W3, b3, CH=8192):
    B = msg.shape[0]
    CH = min(CH, B)
    NCH = B // CH
    F1, F2, F3 = W1.shape[1], W2.shape[1], W3.shape[1]
    Dp = msg.shape[1]
    W1p = jnp.pad(W1, ((0, Dp - W1.shape[0]), (0, 0)))
    return pl.pallas_call(
        functools.partial(_mlp_sa_body, K_NBR),
        grid=(3, NCH),
        in_specs=[
            pl.BlockSpec((CH, Dp), lambda p, j: (j, 0)),
            pl.BlockSpec((Dp, F1), lambda p, j: (0, 0)),
            pl.BlockSpec((1, F1), lambda p, j: (0, 0)),
            pl.BlockSpec((F1, F2), lambda p, j: (0, 0)),
            pl.BlockSpec((1, F2), lambda p, j: (0, 0)),
            pl.BlockSpec((F2, F3), lambda p, j: (0, 0)),
            pl.BlockSpec((1, F3), lambda p, j: (0, 0)),
        ],
        out_specs=pl.BlockSpec((CH // K_NBR, F3), lambda p, j: (j, 0)),
        out_shape=jax.ShapeDtypeStruct((B // K_NBR, F3), jnp.float32),
        scratch_shapes=[pltpu.VMEM((1, F1), jnp.float32),
                        pltpu.VMEM((1, F1), jnp.float32),
                        pltpu.VMEM((1, F2), jnp.float32),
                        pltpu.VMEM((1, F2), jnp.float32),
                        pltpu.VMEM((1, 1), jnp.float32)],
    )(msg, W1p, b1.reshape(1, -1), W2, b2.reshape(1, -1),
      W3, b3.reshape(1, -1))


def _global_body(h_ref, W1_ref, b1_ref, W2_ref, b2_ref, W3_ref, b3_ref,
                 out_ref):
    n = h_ref.shape[0]
    h = h_ref[...]

    def bn(t):
        mean = jnp.sum(t, axis=0, keepdims=True) / n
        var = jnp.sum(t * t, axis=0, keepdims=True) / n - mean * mean
        return (t - mean) / jnp.sqrt(var + EPS)

    h = jax.nn.sigmoid(bn(jnp.dot(h, W1_ref[...],
                                  preferred_element_type=jnp.float32)
                          + b1_ref[...]))
    h = jax.nn.sigmoid(bn(jnp.dot(h, W2_ref[...],
                                  preferred_element_type=jnp.float32)
                          + b2_ref[...]))
    h = jnp.dot(h, W3_ref[...], preferred_element_type=jnp.float32) + b3_ref[...]
    h = jnp.maximum(h, 0.0)
    out = jnp.max(h, axis=0, keepdims=True)
    out = out - jnp.max(out, axis=1, keepdims=True)
    e = jnp.exp(out)
    out_ref[...] = e / jnp.sum(e, axis=1, keepdims=True)


def kernel(x, pos, batch,
           sa1_W1, sa1_b1, sa1_W2, sa1_b2, sa1_W3, sa1_b3,
           sa2_W1, sa2_b1, sa2_W2, sa2_b2, sa2_W3, sa2_b3,
           g_W1, g_b1, g_W2, g_b2, g_W3, g_b3):
    x = (x - jnp.zeros((1, x.shape[1]), x.dtype)) / jnp.ones((1, x.shape[1]), x.dtype)
    x, pos = _sa(x, pos, N1, 1.0, sa1_W1, sa1_b1, sa1_W2, sa1_b2, sa1_W3, sa1_b3)
    x, pos = _sa(x, pos, N2, 2.0, sa2_W1, sa2_b1, sa2_W2, sa2_b2, sa2_W3, sa2_b3)
    h = jnp.concatenate([x, pos], axis=-1)
    Dp = 80
    h = jnp.pad(h, ((0, 0), (0, Dp - h.shape[1])))
    gW1p = jnp.pad(g_W1, ((0, Dp - g_W1.shape[0]), (0, 0)))
    out = pl.pallas_call(
        _global_body,
        out_shape=jax.ShapeDtypeStruct((1, 128), jnp.float32),
    )(h, gW1p, g_b1.reshape(1, -1), g_W2, g_b2.reshape(1, -1),
      g_W3, g_b3.reshape(1, -1))
    return out


# speedup vs baseline: 1.0310x; 1.0310x over previous
import functools

import jax
import jax.numpy as jnp
from jax import lax
from jax.experimental import pallas as pl
from jax.experimental.pallas import tpu as pltpu
from jax.experimental.pallas import tpu_sc as plsc

N_POINTS = 16384
N1 = 4096
N2 = 1024
K_NBR = 32
EPS = 1e-5


def _fps_body(n, px_ref, py_ref, pz_ref, pxs_ref, pys_ref, pzs_ref,
              qx_ref, qy_ref, qz_ref):
    R = px_ref.shape[0]
    px = px_ref[...]
    py = py_ref[...]
    pz = pz_ref[...]
    row_i = lax.broadcasted_iota(jnp.int32, (R, 128), 0)
    col_i = lax.broadcasted_iota(jnp.int32, (R, 128), 1)
    flat_i = row_i * 128 + col_i

    lx0 = pxs_ref[0]
    ly0 = pys_ref[0]
    lz0 = pzs_ref[0]
    qx_ref[0] = lx0
    qy_ref[0] = ly0
    qz_ref[0] = lz0

    dists0 = jnp.full((R, 128), jnp.inf, dtype=jnp.float32)

    def body(i, state):
        dists, lx, ly, lz = state
        dx = px - lx
        dy = py - ly
        dz = pz - lz
        d = (dx * dx + dy * dy) + dz * dz
        dists = jnp.minimum(dists, d)
        m = jnp.max(dists)
        cand = jnp.where(dists == m, flat_i, jnp.int32(1 << 30))
        nxt = jnp.min(cand)
        nlx = pxs_ref[nxt]
        nly = pys_ref[nxt]
        nlz = pzs_ref[nxt]
        qx_ref[i] = nlx
        qy_ref[i] = nly
        qz_ref[i] = nlz
        return (dists, nlx, nly, nlz)

    lax.fori_loop(1, n, body, (dists0, lx0, ly0, lz0), unroll=False)


def _fps_qpos(pos, n):
    N = pos.shape[0]
    R = N // 128
    planes = pos.T.reshape(3, R, 128)
    flat = pos.T
    q = pl.pallas_call(
        functools.partial(_fps_body, n),
        in_specs=[
            pl.BlockSpec(memory_space=pltpu.VMEM),
            pl.BlockSpec(memory_space=pltpu.VMEM),
            pl.BlockSpec(memory_space=pltpu.VMEM),
            pl.BlockSpec(memory_space=pltpu.SMEM),
            pl.BlockSpec(memory_space=pltpu.SMEM),
            pl.BlockSpec(memory_space=pltpu.SMEM),
        ],
        out_specs=[pl.BlockSpec(memory_space=pltpu.SMEM)] * 3,
        out_shape=[jax.ShapeDtypeStruct((n,), jnp.float32)] * 3,
    )(planes[0], planes[1], planes[2], flat[0], flat[1], flat[2])
    return jnp.stack(q, axis=1)


def _topk_body(r2, K, N, q_ref, p_ref, nbr_ref, val_ref, key_ref):
    Q = q_ref.shape[0]
    q8 = q_ref[...]
    p8 = p_ref[...]
    qp = jnp.dot(q8, p8, preferred_element_type=jnp.float32)
    qn = jnp.sum(q8 * q8, axis=1, keepdims=True)
    pn = jnp.sum(p8 * p8, axis=0, keepdims=True)
    d2 = (qn + pn) - 2.0 * qp
    key_ref[...] = jnp.where(d2 <= r2, d2, jnp.inf)
    flat = lax.broadcasted_iota(jnp.int32, (Q, N), 1)
    lane = lax.broadcasted_iota(jnp.int32, (Q, 128), 1)

    def rnd(k, carry):
        nbr_acc, val_acc = carry
        kv = key_ref[...]
        m = jnp.min(kv, axis=1, keepdims=True)
        cand = jnp.where(kv == m, flat, jnp.int32(N))
        nxt = jnp.min(cand, axis=1, keepdims=True)
        key_ref[...] = jnp.where(flat == nxt, jnp.inf, kv)
        nbr_acc = jnp.where(lane == k, nxt, nbr_acc)
        val_acc = jnp.where(lane == k, (m < jnp.inf).astype(jnp.int32), val_acc)
        return nbr_acc, val_acc

    z = jnp.zeros((Q, 128), jnp.int32)
    nbr_acc, val_acc = lax.fori_loop(0, K, rnd, (z, z), unroll=False)
    nbr_ref[...] = nbr_acc
    val_ref[...] = val_acc


def _radius_topk(qpos, pos, r2, Q=256):
    n, N = qpos.shape[0], pos.shape[0]
    q8 = jnp.pad(qpos, ((0, 0), (0, 5)))
    p8 = jnp.pad(pos.T, ((0, 5), (0, 0)))
    grid = n // Q
    nbr, val = pl.pallas_call(
        functools.partial(_topk_body, r2, K_NBR, N),
        grid=(grid,),
        in_specs=[
            pl.BlockSpec((Q, 8), lambda i: (i, 0)),
            pl.BlockSpec((8, N), lambda i: (0, 0)),
        ],
        out_specs=[
            pl.BlockSpec((Q, 128), lambda i: (i, 0)),
            pl.BlockSpec((Q, 128), lambda i: (i, 0)),
        ],
        out_shape=[
            jax.ShapeDtypeStruct((n, 128), jnp.int32),
            jax.ShapeDtypeStruct((n, 128), jnp.int32),
        ],
        scratch_shapes=[pltpu.VMEM((Q, N), jnp.float32)],
    )(q8, p8)
    return nbr[:, :K_NBR], val[:, :K_NBR] != 0


def _sc_gather(table, idx):
    B = idx.shape[0]
    D = table.shape[1]
    info = plsc.get_sparse_core_info()
    NC, NS = info.num_cores, info.num_subcores
    NW = NC * NS
    b_per_w = B // NW
    CH = 128
    n_ch = b_per_w // CH
    idx2d = idx.reshape(B // CH, CH)
    mesh = plsc.VectorSubcoreMesh(core_axis_name="c", subcore_axis_name="s")

    @functools.partial(
        pl.kernel, mesh=mesh,
        compiler_params=pltpu.CompilerParams(use_tc_tiling_on_sc=False),
        out_type=jax.ShapeDtypeStruct((B, D), jnp.float32),
        scratch_types=[
            pltpu.VMEM((n_ch, CH), jnp.int32),
            pltpu.VMEM((b_per_w, D), jnp.float32),
            pltpu.SemaphoreType.DMA,
        ],
    )
    def k(table_hbm, idx_hbm, out_hbm, idx_v, rows_v, sem):
        wid = lax.axis_index("s") * NC + lax.axis_index("c")
        pltpu.sync_copy(idx_hbm.at[pl.ds(wid * n_ch, n_ch)], idx_v)

        def body(j, carry):
            pltpu.async_copy(table_hbm.at[idx_v.at[j]],
                             rows_v.at[pl.ds(j * CH, CH)], sem).wait()
            return carry

        lax.fori_loop(0, n_ch, body, 0)
        pltpu.sync_copy(rows_v, out_hbm.at[pl.ds(wid * b_per_w, b_per_w)])

    return k(table, idx2d)


def _sa(x, pos, n_samples, r, W1, b1, W2, b2, W3, b3):
    qpos = _fps_qpos(pos, n_samples)
    nbr, valid = _radius_topk(qpos, pos, r * r)
    C = x.shape[1]
    Dp = ((C + 3 + 15) // 16) * 16
    table = jnp.pad(jnp.concatenate([x, pos], axis=1),
                    ((0, 0), (0, Dp - C - 3)))
    rows = _sc_gather(table, nbr.reshape(-1))
    xj = rows[:, :C].reshape(n_samples, K_NBR, C)
    posj = rows[:, C:C + 3].reshape(n_samples, K_NBR, 3)
    rel = posj - qpos[:, None, :]
    msg = jnp.concatenate([xj, rel], axis=-1).reshape(n_samples * K_NBR, -1)
    maskf = valid.reshape(n_samples * K_NBR, 1).astype(jnp.float32)
    pad = Dp - msg.shape[1] - 1
    msg = jnp.concatenate(
        [msg, jnp.zeros((msg.shape[0], pad), jnp.float32), maskf], axis=1)
    h = _mlp_sa(msg, W1, b1, W2, b2, W3, b3)
    return h, qpos


def _mlp_sa_body(K, msg_ref, W1_ref, b1_ref, W2_ref, b2_ref,
                 W3_ref, b3_ref, out_ref,
                 s1_ref, q1_ref, s2_ref, q2_ref, sm_ref):
    CH, Dp = msg_ref.shape
    F3 = W3_ref.shape[1]
    p = pl.program_id(0)
    j = pl.program_id(1)

    @pl.when(jnp.logical_and(p == 0, j == 0))
    def _init():
        s1_ref[...] = jnp.zeros_like(s1_ref)
        q1_ref[...] = jnp.zeros_like(q1_ref)
        s2_ref[...] = jnp.zeros_like(s2_ref)
        q2_ref[...] = jnp.zeros_like(q2_ref)
        sm_ref[...] = jnp.zeros_like(sm_ref)

    mc = msg_ref[...]
    mk = mc[:, Dp - 1:Dp]

    @pl.when(p == 0)
    def _ph1():
        pre = jnp.dot(mc, W1_ref[...], preferred_element_type=jnp.float32) \
            + b1_ref[...]
        wp = mk * pre
        s1_ref[...] += jnp.sum(wp, axis=0, keepdims=True)
        q1_ref[...] += jnp.sum(wp * pre, axis=0, keepdims=True)
        sm_ref[...] += jnp.sum(mk, keepdims=True).reshape(1, 1)

    @pl.when(p > 0)
    def _ph23():
        sm = sm_ref[0, 0]
        mean1 = s1_ref[...] / sm
        inv1 = 1.0 / jnp.sqrt(q1_ref[...] / sm - mean1 * mean1 + EPS)
        pre1 = jnp.dot(mc, W1_ref[...], preferred_element_type=jnp.float32) \
            + b1_ref[...]
        h1 = jax.nn.sigmoid((pre1 - mean1) * inv1)
        pre2 = jnp.dot(h1, W2_ref[...], preferred_element_type=jnp.float32) \
            + b2_ref[...]

        @pl.when(p == 1)
        def _acc2():
            wp = mk * pre2
            s2_ref[...] += jnp.sum(wp, axis=0, keepdims=True)
            q2_ref[...] += jnp.sum(wp * pre2, axis=0, keepdims=True)

        @pl.when(p == 2)
        def _ph3():
            mean2 = s2_ref[...] / sm
            inv2 = 1.0 / jnp.sqrt(q2_ref[...] / sm - mean2 * mean2 + EPS)
            h2 = jax.nn.sigmoid((pre2 - mean2) * inv2)
            h3 = jnp.dot(h2, W3_ref[...],
                         preferred_element_type=jnp.float32) + b3_ref[...]
            h3 = jnp.where(mk > 0.0, h3, -jnp.inf)
            pooled = jnp.max(h3.reshape(CH // K, K, F3), axis=1)
            out_ref[...] = jnp.maximum(pooled, 0.0)


def _mlp_sa(msg, W1, b1, W2, b2, W3, b3, CH=8192):
    B = msg.shape[0]
    CH = min(CH, B)
    NCH = B // CH
    F1, F2, F3 = W1.shape[1], W2.shape[1], W3.shape[1]
    Dp = msg.shape[1]
    W1p = jnp.pad(W1, ((0, Dp - W1.shape[0]), (0, 0)))
    return pl.pallas_call(
        functools.partial(_mlp_sa_body, K_NBR),
        grid=(3, NCH),
        in_specs=[
            pl.BlockSpec((CH, Dp), lambda p, j: (j, 0)),
            pl.BlockSpec((Dp, F1), lambda p, j: (0, 0)),
            pl.BlockSpec((1, F1), lambda p, j: (0, 0)),
            pl.BlockSpec((F1, F2), lambda p, j: (0, 0)),
            pl.BlockSpec((1, F2), lambda p, j: (0, 0)),
            pl.BlockSpec((F2, F3), lambda p, j: (0, 0)),
            pl.BlockSpec((1, F3), lambda p, j: (0, 0)),
        ],
        out_specs=pl.BlockSpec((CH // K_NBR, F3), lambda p, j: (j, 0)),
        out_shape=jax.ShapeDtypeStruct((B // K_NBR, F3), jnp.float32),
        scratch_shapes=[pltpu.VMEM((1, F1), jnp.float32),
                        pltpu.VMEM((1, F1), jnp.float32),
                        pltpu.VMEM((1, F2), jnp.float32),
                        pltpu.VMEM((1, F2), jnp.float32),
                        pltpu.VMEM((1, 1), jnp.float32)],
    )(msg, W1p, b1.reshape(1, -1), W2, b2.reshape(1, -1),
      W3, b3.reshape(1, -1))


def _global_body(h_ref, W1_ref, b1_ref, W2_ref, b2_ref, W3_ref, b3_ref,
                 out_ref):
    n = h_ref.shape[0]
    h = h_ref[...]

    def bn(t):
        mean = jnp.sum(t, axis=0, keepdims=True) / n
        var = jnp.sum(t * t, axis=0, keepdims=True) / n - mean * mean
        return (t - mean) / jnp.sqrt(var + EPS)

    h = jax.nn.sigmoid(bn(jnp.dot(h, W1_ref[...],
                                  preferred_element_type=jnp.float32)
                          + b1_ref[...]))
    h = jax.nn.sigmoid(bn(jnp.dot(h, W2_ref[...],
                                  preferred_element_type=jnp.float32)
                          + b2_ref[...]))
    h = jnp.dot(h, W3_ref[...], preferred_element_type=jnp.float32) + b3_ref[...]
    h = jnp.maximum(h, 0.0)
    out = jnp.max(h, axis=0, keepdims=True)
    out = out - jnp.max(out, axis=1, keepdims=True)
    e = jnp.exp(out)
    out_ref[...] = e / jnp.sum(e, axis=1, keepdims=True)


def kernel(x, pos, batch,
           sa1_W1, sa1_b1, sa1_W2, sa1_b2, sa1_W3, sa1_b3,
           sa2_W1, sa2_b1, sa2_W2, sa2_b2, sa2_W3, sa2_b3,
           g_W1, g_b1, g_W2, g_b2, g_W3, g_b3):
    x = (x - jnp.zeros((1, x.shape[1]), x.dtype)) / jnp.ones((1, x.shape[1]), x.dtype)
    x, pos = _sa(x, pos, N1, 1.0, sa1_W1, sa1_b1, sa1_W2, sa1_b2, sa1_W3, sa1_b3)
    x, pos = _sa(x, pos, N2, 2.0, sa2_W1, sa2_b1, sa2_W2, sa2_b2, sa2_W3, sa2_b3)
    h = jnp.concatenate([x, pos], axis=-1)
    Dp = 80
    h = jnp.pad(h, ((0, 0), (0, Dp - h.shape[1])))
    gW1p = jnp.pad(g_W1, ((0, Dp - g_W1.shape[0]), (0, 0)))
    out = pl.pallas_call(
        _global_body,
        out_shape=jax.ShapeDtypeStruct((1, 128), jnp.float32),
    )(h, gW1p, g_b1.reshape(1, -1), g_W2, g_b2.reshape(1, -1),
      g_W3, g_b3.reshape(1, -1))
    return out


# speedup vs baseline: 1.0479x; 1.0163x over previous
import functools

import jax
import jax.numpy as jnp
from jax import lax
from jax.experimental import pallas as pl
from jax.experimental.pallas import tpu as pltpu
from jax.experimental.pallas import tpu_sc as plsc

N_POINTS = 16384
N1 = 4096
N2 = 1024
K_NBR = 32
EPS = 1e-5


def _fps_body(n, px_ref, py_ref, pz_ref, pxs_ref, pys_ref, pzs_ref,
              qx_ref, qy_ref, qz_ref):
    R = px_ref.shape[0]
    px = px_ref[...]
    py = py_ref[...]
    pz = pz_ref[...]
    row_i = lax.broadcasted_iota(jnp.int32, (R, 128), 0)
    col_i = lax.broadcasted_iota(jnp.int32, (R, 128), 1)
    flat_i = row_i * 128 + col_i

    lx0 = pxs_ref[0]
    ly0 = pys_ref[0]
    lz0 = pzs_ref[0]
    qx_ref[0] = lx0
    qy_ref[0] = ly0
    qz_ref[0] = lz0

    dists0 = jnp.full((R, 128), jnp.inf, dtype=jnp.float32)

    def body(i, state):
        dists, lx, ly, lz = state
        dx = px - lx
        dy = py - ly
        dz = pz - lz
        d = (dx * dx + dy * dy) + dz * dz
        dists = jnp.minimum(dists, d)
        m = jnp.max(dists)
        cand = jnp.where(dists == m, flat_i, jnp.int32(1 << 30))
        nxt = jnp.min(cand)
        nlx = pxs_ref[nxt]
        nly = pys_ref[nxt]
        nlz = pzs_ref[nxt]
        qx_ref[i] = nlx
        qy_ref[i] = nly
        qz_ref[i] = nlz
        return (dists, nlx, nly, nlz)

    lax.fori_loop(1, n, body, (dists0, lx0, ly0, lz0), unroll=False)


def _fps_qpos(pos, n):
    N = pos.shape[0]
    R = N // 128
    planes = pos.T.reshape(3, R, 128)
    flat = pos.T
    q = pl.pallas_call(
        functools.partial(_fps_body, n),
        in_specs=[
            pl.BlockSpec(memory_space=pltpu.VMEM),
            pl.BlockSpec(memory_space=pltpu.VMEM),
            pl.BlockSpec(memory_space=pltpu.VMEM),
            pl.BlockSpec(memory_space=pltpu.SMEM),
            pl.BlockSpec(memory_space=pltpu.SMEM),
            pl.BlockSpec(memory_space=pltpu.SMEM),
        ],
        out_specs=[pl.BlockSpec(memory_space=pltpu.SMEM)] * 3,
        out_shape=[jax.ShapeDtypeStruct((n,), jnp.float32)] * 3,
    )(planes[0], planes[1], planes[2], flat[0], flat[1], flat[2])
    return jnp.stack(q, axis=1)


def _topk_body(r2, K, N, q_ref, p_ref, nbr_ref, val_ref, key_ref):
    Q = q_ref.shape[0]
    q8 = q_ref[...]
    p8 = p_ref[...]
    qp = jnp.dot(q8, p8, preferred_element_type=jnp.float32)
    qn = jnp.sum(q8 * q8, axis=1, keepdims=True)
    pn = jnp.sum(p8 * p8, axis=0, keepdims=True)
    d2 = (qn + pn) - 2.0 * qp
    key_ref[...] = jnp.where(d2 <= r2, d2, jnp.inf)
    flat = lax.broadcasted_iota(jnp.int32, (Q, N), 1)
    lane = lax.broadcasted_iota(jnp.int32, (Q, 128), 1)

    def rnd(k, carry):
        nbr_acc, val_acc = carry
        kv = key_ref[...]
        m = jnp.min(kv, axis=1, keepdims=True)
        cand = jnp.where(kv == m, flat, jnp.int32(N))
        nxt = jnp.min(cand, axis=1, keepdims=True)
        key_ref[...] = jnp.where(flat == nxt, jnp.inf, kv)
        nbr_acc = jnp.where(lane == k, nxt, nbr_acc)
        val_acc = jnp.where(lane == k, (m < jnp.inf).astype(jnp.int32), val_acc)
        return nbr_acc, val_acc

    z = jnp.zeros((Q, 128), jnp.int32)
    nbr_acc, val_acc = lax.fori_loop(0, K, rnd, (z, z), unroll=False)
    nbr_ref[...] = nbr_acc
    val_ref[...] = val_acc


def _radius_topk(qpos, pos, r2, Q=512):
    n, N = qpos.shape[0], pos.shape[0]
    q8 = jnp.pad(qpos, ((0, 0), (0, 5)))
    p8 = jnp.pad(pos.T, ((0, 5), (0, 0)))
    grid = n // Q
    nbr, val = pl.pallas_call(
        functools.partial(_topk_body, r2, K_NBR, N),
        grid=(grid,),
        in_specs=[
            pl.BlockSpec((Q, 8), lambda i: (i, 0)),
            pl.BlockSpec((8, N), lambda i: (0, 0)),
        ],
        out_specs=[
            pl.BlockSpec((Q, 128), lambda i: (i, 0)),
            pl.BlockSpec((Q, 128), lambda i: (i, 0)),
        ],
        out_shape=[
            jax.ShapeDtypeStruct((n, 128), jnp.int32),
            jax.ShapeDtypeStruct((n, 128), jnp.int32),
        ],
        scratch_shapes=[pltpu.VMEM((Q, N), jnp.float32)],
    )(q8, p8)
    return nbr[:, :K_NBR], val[:, :K_NBR] != 0


def _sc_gather(table, idx):
    B = idx.shape[0]
    D = table.shape[1]
    info = plsc.get_sparse_core_info()
    NC, NS = info.num_cores, info.num_subcores
    NW = NC * NS
    b_per_w = B // NW
    CH = 128
    n_ch = b_per_w // CH
    idx2d = idx.reshape(B // CH, CH)
    mesh = plsc.VectorSubcoreMesh(core_axis_name="c", subcore_axis_name="s")

    @functools.partial(
        pl.kernel, mesh=mesh,
        compiler_params=pltpu.CompilerParams(use_tc_tiling_on_sc=False),
        out_type=jax.ShapeDtypeStruct((B, D), jnp.float32),
        scratch_types=[
            pltpu.VMEM((n_ch, CH), jnp.int32),
            pltpu.VMEM((b_per_w, D), jnp.float32),
            pltpu.SemaphoreType.DMA,
        ],
    )
    def k(table_hbm, idx_hbm, out_hbm, idx_v, rows_v, sem):
        wid = lax.axis_index("s") * NC + lax.axis_index("c")
        pltpu.sync_copy(idx_hbm.at[pl.ds(wid * n_ch, n_ch)], idx_v)

        def body(j, carry):
            pltpu.async_copy(table_hbm.at[idx_v.at[j]],
                             rows_v.at[pl.ds(j * CH, CH)], sem).wait()
            return carry

        lax.fori_loop(0, n_ch, body, 0)
        pltpu.sync_copy(rows_v, out_hbm.at[pl.ds(wid * b_per_w, b_per_w)])

    return k(table, idx2d)


def _sa(x, pos, n_samples, r, W1, b1, W2, b2, W3, b3):
    qpos = _fps_qpos(pos, n_samples)
    nbr, valid = _radius_topk(qpos, pos, r * r)
    C = x.shape[1]
    Dp = ((C + 3 + 15) // 16) * 16
    table = jnp.pad(jnp.concatenate([x, pos], axis=1),
                    ((0, 0), (0, Dp - C - 3)))
    rows = _sc_gather(table, nbr.reshape(-1))
    xj = rows[:, :C].reshape(n_samples, K_NBR, C)
    posj = rows[:, C:C + 3].reshape(n_samples, K_NBR, 3)
    rel = posj - qpos[:, None, :]
    msg = jnp.concatenate([xj, rel], axis=-1).reshape(n_samples * K_NBR, -1)
    maskf = valid.reshape(n_samples * K_NBR, 1).astype(jnp.float32)
    pad = Dp - msg.shape[1] - 1
    msg = jnp.concatenate(
        [msg, jnp.zeros((msg.shape[0], pad), jnp.float32), maskf], axis=1)
    h = _mlp_sa(msg, W1, b1, W2, b2, W3, b3)
    return h, qpos


def _mlp_sa_body(K, msg_ref, W1_ref, b1_ref, W2_ref, b2_ref,
                 W3_ref, b3_ref, out_ref,
                 s1_ref, q1_ref, s2_ref, q2_ref, sm_ref):
    CH, Dp = msg_ref.shape
    F3 = W3_ref.shape[1]
    p = pl.program_id(0)
    j = pl.program_id(1)

    @pl.when(jnp.logical_and(p == 0, j == 0))
    def _init():
        s1_ref[...] = jnp.zeros_like(s1_ref)
        q1_ref[...] = jnp.zeros_like(q1_ref)
        s2_ref[...] = jnp.zeros_like(s2_ref)
        q2_ref[...] = jnp.zeros_like(q2_ref)
        sm_ref[...] = jnp.zeros_like(sm_ref)

    mc = msg_ref[...]
    mk = mc[:, Dp - 1:Dp]

    @pl.when(p == 0)
    def _ph1():
        pre = jnp.dot(mc, W1_ref[...], preferred_element_type=jnp.float32) \
            + b1_ref[...]
        wp = mk * pre
        s1_ref[...] += jnp.sum(wp, axis=0, keepdims=True)
        q1_ref[...] += jnp.sum(wp * pre, axis=0, keepdims=True)
        sm_ref[...] += jnp.sum(mk, keepdims=True).reshape(1, 1)

    @pl.when(p > 0)
    def _ph23():
        sm = sm_ref[0, 0]
        mean1 = s1_ref[...] / sm
        inv1 = 1.0 / jnp.sqrt(q1_ref[...] / sm - mean1 * mean1 + EPS)
        pre1 = jnp.dot(mc, W1_ref[...], preferred_element_type=jnp.float32) \
            + b1_ref[...]
        h1 = jax.nn.sigmoid((pre1 - mean1) * inv1)
        pre2 = jnp.dot(h1, W2_ref[...], preferred_element_type=jnp.float32) \
            + b2_ref[...]

        @pl.when(p == 1)
        def _acc2():
            wp = mk * pre2
            s2_ref[...] += jnp.sum(wp, axis=0, keepdims=True)
            q2_ref[...] += jnp.sum(wp * pre2, axis=0, keepdims=True)

        @pl.when(p == 2)
        def _ph3():
            mean2 = s2_ref[...] / sm
            inv2 = 1.0 / jnp.sqrt(q2_ref[...] / sm - mean2 * mean2 + EPS)
            h2 = jax.nn.sigmoid((pre2 - mean2) * inv2)
            h3 = jnp.dot(h2, W3_ref[...],
                         preferred_element_type=jnp.float32) + b3_ref[...]
            h3 = jnp.where(mk > 0.0, h3, -jnp.inf)
            pooled = jnp.max(h3.reshape(CH // K, K, F3), axis=1)
            out_ref[...] = jnp.maximum(pooled, 0.0)


def _mlp_sa(msg, W1, b1, W2, b2, W3, b3, CH=8192):
    B = msg.shape[0]
    CH = min(CH, B)
    NCH = B // CH
    F1, F2, F3 = W1.shape[1], W2.shape[1], W3.shape[1]
    Dp = msg.shape[1]
    W1p = jnp.pad(W1, ((0, Dp - W1.shape[0]), (0, 0)))
    return pl.pallas_call(
        functools.partial(_mlp_sa_body, K_NBR),
        grid=(3, NCH),
        in_specs=[
            pl.BlockSpec((CH, Dp), lambda p, j: (j, 0)),
            pl.BlockSpec((Dp, F1), lambda p, j: (0, 0)),
            pl.BlockSpec((1, F1), lambda p, j: (0, 0)),
            pl.BlockSpec((F1, F2), lambda p, j: (0, 0)),
            pl.BlockSpec((1, F2), lambda p, j: (0, 0)),
            pl.BlockSpec((F2, F3), lambda p, j: (0, 0)),
            pl.BlockSpec((1, F3), lambda p, j: (0, 0)),
        ],
        out_specs=pl.BlockSpec((CH // K_NBR, F3), lambda p, j: (j, 0)),
        out_shape=jax.ShapeDtypeStruct((B // K_NBR, F3), jnp.float32),
        scratch_shapes=[pltpu.VMEM((1, F1), jnp.float32),
                        pltpu.VMEM((1, F1), jnp.float32),
                        pltpu.VMEM((1, F2), jnp.float32),
                        pltpu.VMEM((1, F2), jnp.float32),
                        pltpu.VMEM((1, 1), jnp.float32)],
    )(msg, W1p, b1.reshape(1, -1), W2, b2.reshape(1, -1),
      W3, b3.reshape(1, -1))


def _global_body(h_ref, W1_ref, b1_ref, W2_ref, b2_ref, W3_ref, b3_ref,
                 out_ref):
    n = h_ref.shape[0]
    h = h_ref[...]

    def bn(t):
        mean = jnp.sum(t, axis=0, keepdims=True) / n
        var = jnp.sum(t * t, axis=0, keepdims=True) / n - mean * mean
        return (t - mean) / jnp.sqrt(var + EPS)

    h = jax.nn.sigmoid(bn(jnp.dot(h, W1_ref[...],
                                  preferred_element_type=jnp.float32)
                          + b1_ref[...]))
    h = jax.nn.sigmoid(bn(jnp.dot(h, W2_ref[...],
                                  preferred_element_type=jnp.float32)
                          + b2_ref[...]))
    h = jnp.dot(h, W3_ref[...], preferred_element_type=jnp.float32) + b3_ref[...]
    h = jnp.maximum(h, 0.0)
    out = jnp.max(h, axis=0, keepdims=True)
    out = out - jnp.max(out, axis=1, keepdims=True)
    e = jnp.exp(out)
    out_ref[...] = e / jnp.sum(e, axis=1, keepdims=True)


def kernel(x, pos, batch,
           sa1_W1, sa1_b1, sa1_W2, sa1_b2, sa1_W3, sa1_b3,
           sa2_W1, sa2_b1, sa2_W2, sa2_b2, sa2_W3, sa2_b3,
           g_W1, g_b1, g_W2, g_b2, g_W3, g_b3):
    x = (x - jnp.zeros((1, x.shape[1]), x.dtype)) / jnp.ones((1, x.shape[1]), x.dtype)
    x, pos = _sa(x, pos, N1, 1.0, sa1_W1, sa1_b1, sa1_W2, sa1_b2, sa1_W3, sa1_b3)
    x, pos = _sa(x, pos, N2, 2.0, sa2_W1, sa2_b1, sa2_W2, sa2_b2, sa2_W3, sa2_b3)
    h = jnp.concatenate([x, pos], axis=-1)
    Dp = 80
    h = jnp.pad(h, ((0, 0), (0, Dp - h.shape[1])))
    gW1p = jnp.pad(g_W1, ((0, Dp - g_W1.shape[0]), (0, 0)))
    out = pl.pallas_call(
        _global_body,
        out_shape=jax.ShapeDtypeStruct((1, 128), jnp.float32),
    )(h, gW1p, g_b1.reshape(1, -1), g_W2, g_b2.reshape(1, -1),
      g_W3, g_b3.reshape(1, -1))
    return out
